# baseline (speedup 1.0000x reference)
import functools

import jax
import jax.numpy as jnp
from jax import lax
from jax.experimental import pallas as pl
from jax.experimental.pallas import tpu as pltpu
from jax.experimental.pallas import tpu_sc as plsc

N = 10000
D = 128
E = 320000
NC = 2
NS = 16
NW = NC * NS
C = 128
NCH = 80
EPW = C * NCH
E_PAD = NW * EPW
N_ACC = 10240
RPT = N_ACC // NS
F32 = jnp.float32

_mesh = plsc.VectorSubcoreMesh(core_axis_name="c", subcore_axis_name="s")


@functools.partial(
    pl.kernel,
    out_type=jax.ShapeDtypeStruct((NC, N_ACC, D), F32),
    mesh=_mesh,
    scratch_types=[
        pltpu.VMEM((NCH, C), jnp.int32),
        pltpu.VMEM((2, C), jnp.int32),
        pltpu.VMEM((2, C), jnp.int32),
        pltpu.VMEM((C, D), F32),
        pltpu.VMEM((C, D), F32),
        pltpu.VMEM_SHARED((N_ACC, D), F32),
        pltpu.SemaphoreType.DMA,
        pltpu.SemaphoreType.DMA,
    ],
)
def _agg(table, pk_hbm, out_p,
         pkv, srow, drow, buf0, buf1, acc, sem0, sem1):
    c = lax.axis_index("c")
    s = lax.axis_index("s")
    wid = c * NS + s

    pltpu.sync_copy(pk_hbm.at[wid], pkv)

    def unpack_src(ch, b):
        for k in range(8):
            v = pkv[ch, pl.ds(16 * k, 16)]
            srow[b, pl.ds(16 * k, 16)] = lax.shift_right_logical(v, 14)

    def unpack_dst(ch, b):
        for k in range(8):
            v = pkv[ch, pl.ds(16 * k, 16)]
            drow[b, pl.ds(16 * k, 16)] = lax.bitwise_and(v, 16383)

    zv = jnp.zeros((16,), F32)

    def zrow(r, carry):
        for k in range(8):
            buf0[r, pl.ds(16 * k, 16)] = zv
        return carry

    lax.fori_loop(0, C, zrow, 0)
    row0 = s * RPT
    for t in range(RPT // C):
        pltpu.sync_copy(buf0, acc.at[pl.ds(row0 + t * C, C)])

    plsc.subcore_barrier()

    bufs = (buf0, buf1)
    sems = (sem0, sem1)
    for b in range(2):
        unpack_src(b, b)
        pltpu.async_copy(table.at[srow.at[b]], bufs[b], sems[b])

    def chunk_pair(i, carry):
        j = i * 2
        for b in range(2):
            ch = j + b
            pltpu.make_async_copy(table.at[srow.at[b]], bufs[b], sems[b]).wait()
            unpack_dst(ch, b)
            pltpu.sync_copy(bufs[b], acc.at[drow.at[b]], add=True)
            nxt = ch + 2

            @pl.when(nxt < NCH)
            def _():
                unpack_src(nxt, b)
                pltpu.async_copy(table.at[srow.at[b]], bufs[b], sems[b])

        return carry

    lax.fori_loop(0, NCH // 2, chunk_pair, 0)
    plsc.subcore_barrier()

    for t in range(RPT // C):
        r0 = row0 + t * C
        pltpu.sync_copy(acc.at[pl.ds(r0, C)], buf0)
        pltpu.sync_copy(buf0, out_p.at[c, pl.ds(r0, C)])


@functools.partial(
    pl.kernel,
    out_type=jax.ShapeDtypeStruct((NC, N_ACC, D), F32),
    mesh=_mesh,
    scratch_types=[
        pltpu.VMEM((NCH, C), jnp.int32),
        pltpu.VMEM((1, C), jnp.int32),
        pltpu.VMEM((C, D), F32),
        pltpu.VMEM_SHARED((N_ACC, D), F32),
    ],
)
def _cnt(pk_hbm, out_cnt, pkv, drow, ones, cnt_acc):
    c = lax.axis_index("c")
    s = lax.axis_index("s")
    wid = c * NS + s

    pltpu.sync_copy(pk_hbm.at[wid], pkv)

    zv = jnp.zeros((16,), F32)

    def zrow(r, carry):
        for k in range(8):
            ones[r, pl.ds(16 * k, 16)] = zv
        return carry

    lax.fori_loop(0, C, zrow, 0)
    row0 = s * RPT
    for t in range(RPT // C):
        pltpu.sync_copy(ones, cnt_acc.at[pl.ds(row0 + t * C, C)])
    ov = jnp.ones((16,), F32)

    def orow(r, carry):
        for k in range(8):
            ones[r, pl.ds(16 * k, 16)] = ov
        return carry

    lax.fori_loop(0, C, orow, 0)
    plsc.subcore_barrier()

    def chunk(ch, carry):
        for k in range(8):
            v = pkv[ch, pl.ds(16 * k, 16)]
            drow[0, pl.ds(16 * k, 16)] = lax.bitwise_and(v, 16383)
        pltpu.sync_copy(ones, cnt_acc.at[drow.at[0]], add=True)
        return carry

    lax.fori_loop(0, NCH, chunk, 0)
    plsc.subcore_barrier()

    for t in range(RPT // C):
        r0 = row0 + t * C
        pltpu.sync_copy(cnt_acc.at[pl.ds(r0, C)], ones)
        pltpu.sync_copy(ones, out_cnt.at[c, pl.ds(r0, C)])


def _mk_exp(random_gather, random_scatter):
    @functools.partial(
        pl.kernel,
        out_type=jax.ShapeDtypeStruct((NC, N_ACC, D), F32),
        mesh=_mesh,
        scratch_types=[
            pltpu.VMEM((NCH, C), jnp.int32),
            pltpu.VMEM((2, C), jnp.int32),
            pltpu.VMEM((2, C), jnp.int32),
            pltpu.VMEM((C, D), F32),
            pltpu.VMEM((C, D), F32),
            pltpu.VMEM_SHARED((N_ACC, D), F32),
            pltpu.SemaphoreType.DMA,
            pltpu.SemaphoreType.DMA,
        ],
    )
    def _exp(table, pk_hbm, out_p, pkv, srow, drow, buf0, buf1, acc, sem0, sem1):
        c = lax.axis_index("c")
        s = lax.axis_index("s")
        wid = c * NS + s
        pltpu.sync_copy(pk_hbm.at[wid], pkv)
        row0 = s * RPT
        iota = lax.iota(jnp.int32, 16)

        def unpack_src(ch, b):
            for k in range(8):
                if random_gather:
                    v = pkv[ch, pl.ds(16 * k, 16)]
                    srow[b, pl.ds(16 * k, 16)] = lax.shift_right_logical(v, 14)
                else:
                    srow[b, pl.ds(16 * k, 16)] = iota + (16 * k + row0)

        def unpack_dst(ch, b):
            for k in range(8):
                if random_scatter:
                    v = pkv[ch, pl.ds(16 * k, 16)]
                    drow[b, pl.ds(16 * k, 16)] = lax.bitwise_and(v, 16383)
                else:
                    drow[b, pl.ds(16 * k, 16)] = iota + (16 * k + row0)

        zv = jnp.zeros((16,), F32)

        def zrow(r, carry):
            for k in range(8):
                buf0[r, pl.ds(16 * k, 16)] = zv
            return carry

        lax.fori_loop(0, C, zrow, 0)
        for t in range(RPT // C):
            pltpu.sync_copy(buf0, acc.at[pl.ds(row0 + t * C, C)])
        plsc.subcore_barrier()
        bufs = (buf0, buf1)
        sems = (sem0, sem1)
        for b in range(2):
            unpack_src(b, b)
            pltpu.async_copy(table.at[srow.at[b]], bufs[b], sems[b])

        def chunk_pair(i, carry):
            j = i * 2
            for b in range(2):
                ch = j + b
                pltpu.make_async_copy(table.at[srow.at[b]], bufs[b], sems[b]).wait()
                unpack_dst(ch, b)
                pltpu.sync_copy(bufs[b], acc.at[drow.at[b]], add=True)
                nxt = ch + 2

                @pl.when(nxt < NCH)
                def _():
                    unpack_src(nxt, b)
                    pltpu.async_copy(table.at[srow.at[b]], bufs[b], sems[b])

            return carry

        lax.fori_loop(0, NCH // 2, chunk_pair, 0)
        plsc.subcore_barrier()
        for t in range(RPT // C):
            r0 = row0 + t * C
            pltpu.sync_copy(acc.at[pl.ds(r0, C)], buf0)
            pltpu.sync_copy(buf0, out_p.at[c, pl.ds(r0, C)])
    return _exp


_exp_gonly = _mk_exp(True, False)
_exp_sonly = _mk_exp(False, True)


_R = 2048


def _tc1_body(p_ref, cnt_ref, x_ref, wlt_ref, bl_ref, wrt_ref,
              bns_ref, bnb_ref, h_ref):
    sagg = p_ref[0] + p_ref[1]
    cnt = (cnt_ref[0] + cnt_ref[1])[:, 0:1]
    mean = sagg / jnp.maximum(cnt, 1.0)
    h = jnp.dot(mean, wlt_ref[...], preferred_element_type=F32) + bl_ref[...]
    h = h + jnp.dot(x_ref[...], wrt_ref[...], preferred_element_type=F32)
    h = h * bns_ref[...] + bnb_ref[...]
    h_ref[...] = jnp.maximum(h, 0.0)


_tc1 = pl.pallas_call(
    _tc1_body,
    grid=(N_ACC // _R,),
    in_specs=[
        pl.BlockSpec((NC, _R, D), lambda i: (0, i, 0)),
        pl.BlockSpec((NC, _R, D), lambda i: (0, i, 0)),
        pl.BlockSpec((_R, D), lambda i: (i, 0)),
        pl.BlockSpec((D, D), lambda i: (0, 0)),
        pl.BlockSpec((1, D), lambda i: (0, 0)),
        pl.BlockSpec((D, D), lambda i: (0, 0)),
        pl.BlockSpec((1, D), lambda i: (0, 0)),
        pl.BlockSpec((1, D), lambda i: (0, 0)),
    ],
    out_specs=pl.BlockSpec((_R, D), lambda i: (i, 0)),
    out_shape=jax.ShapeDtypeStruct((N_ACC, D), F32),
)


def _tc2_body(p_ref, cnt_ref, h1_ref, wlt_ref, bl_ref, wrt_ref,
              bns_ref, bnb_ref, wc1t_ref, bc1_ref, wc2t_ref, bc2_ref,
              lg_ref, pr_ref):
    sagg = p_ref[0] + p_ref[1]
    cnt = (cnt_ref[0] + cnt_ref[1])[:, 0:1]
    mean = sagg / jnp.maximum(cnt, 1.0)
    h = jnp.dot(mean, wlt_ref[...], preferred_element_type=F32) + bl_ref[...]
    h = h + jnp.dot(h1_ref[...], wrt_ref[...], preferred_element_type=F32)
    h = jnp.maximum(h * bns_ref[...] + bnb_ref[...], 0.0)
    z = jnp.maximum(
        jnp.dot(h, wc1t_ref[...], preferred_element_type=F32) + bc1_ref[...],
        0.0)
    lg = jnp.dot(z, wc2t_ref[...], preferred_element_type=F32) + bc2_ref[...]
    m = jnp.max(lg, axis=-1, keepdims=True)
    ex = jnp.exp(lg - m)
    pr = ex / jnp.sum(ex, axis=-1, keepdims=True)
    lg_ref[...] = lg
    pr_ref[...] = pr


_tc2 = pl.pallas_call(
    _tc2_body,
    grid=(N_ACC // _R,),
    in_specs=[
        pl.BlockSpec((NC, _R, D), lambda i: (0, i, 0)),
        pl.BlockSpec((NC, _R, D), lambda i: (0, i, 0)),
        pl.BlockSpec((_R, D), lambda i: (i, 0)),
        pl.BlockSpec((D, D), lambda i: (0, 0)),
        pl.BlockSpec((1, D), lambda i: (0, 0)),
        pl.BlockSpec((D, D), lambda i: (0, 0)),
        pl.BlockSpec((1, D), lambda i: (0, 0)),
        pl.BlockSpec((1, D), lambda i: (0, 0)),
        pl.BlockSpec((D, 64), lambda i: (0, 0)),
        pl.BlockSpec((1, 64), lambda i: (0, 0)),
        pl.BlockSpec((64, D), lambda i: (0, 0)),
        pl.BlockSpec((1, D), lambda i: (0, 0)),
    ],
    out_specs=(pl.BlockSpec((_R, D), lambda i: (i, 0)),
               pl.BlockSpec((_R, D), lambda i: (i, 0))),
    out_shape=(jax.ShapeDtypeStruct((N_ACC, D), F32),
               jax.ShapeDtypeStruct((N_ACC, D), F32)),
)


def kernel(x, edge_index, Wl1, bl1, Wr1, g1, be1,
           Wl2, bl2, Wr2, g2, be2, Wc1, bc1, Wc2, bc2):
    src = edge_index[0]
    dst = edge_index[1]
    pad = E_PAD - E
    packed = src * 16384 + dst
    pk = jnp.concatenate(
        [packed, jnp.full((pad,), N, jnp.int32)]).reshape(NW, NCH, C)
    x_pad = jnp.zeros((N_ACC, D), F32).at[:N].set(x)

    p1 = _agg(x_pad, pk)
    cnt16 = _cnt(pk)

    bscale = 1.0 / jnp.sqrt(jnp.float32(1.0 + 1e-5))
    h1 = _tc1(p1, cnt16, x_pad, Wl1.T, bl1.reshape(1, D), Wr1.T,
              (g1 * bscale).reshape(1, D), be1.reshape(1, D))

    p2 = _agg(h1, pk)

    wc2t = jnp.zeros((64, D), F32).at[:, :2].set(Wc2.T)
    bc2p = jnp.full((1, D), -1e30, F32).at[0, :2].set(bc2)
    logits_pad, probs_pad = _tc2(
        p2, cnt16, h1, Wl2.T, bl2.reshape(1, D), Wr2.T,
        (g2 * bscale).reshape(1, D), be2.reshape(1, D),
        Wc1.T, bc1.reshape(1, 64), wc2t, bc2p)

    eg = _exp_gonly(x_pad, pk)
    es = _exp_sonly(x_pad, pk)
    eps = 1e-30 * (eg[0, :N, :2] + es[0, :N, :2])
    return (logits_pad[:N, :2] + eps, probs_pad[:N, :2])


# speedup vs baseline: 2.1472x; 2.1472x over previous
import functools

import jax
import jax.numpy as jnp
from jax import lax
from jax.experimental import pallas as pl
from jax.experimental.pallas import tpu as pltpu
from jax.experimental.pallas import tpu_sc as plsc

N = 10000
D = 128
E = 320000
NC = 2
NS = 16
NW = NC * NS
C = 128
NCH = 80
EPW = C * NCH
E_PAD = NW * EPW
N_ACC = 10240
RPT = N_ACC // NS
DH = D // 2
NCHT = 160
F32 = jnp.float32

_mesh = plsc.VectorSubcoreMesh(core_axis_name="c", subcore_axis_name="s")


K0 = 120
K1 = NCHT - K0
KMAX = max(K0, K1)


@functools.partial(
    pl.kernel,
    out_type=jax.ShapeDtypeStruct((NC, N_ACC, D), F32),
    mesh=_mesh,
    scratch_types=[
        pltpu.VMEM((KMAX, C), jnp.int32),
        pltpu.VMEM((2, C), jnp.int32),
        pltpu.VMEM((2, C), jnp.int32),
        pltpu.VMEM((C, D), F32),
        pltpu.VMEM((C, D), F32),
        pltpu.VMEM_SHARED((N_ACC, D), F32),
        pltpu.SemaphoreType.DMA,
        pltpu.SemaphoreType.DMA,
    ],
)
def _agg(table, pk_hbm, out_p,
         pkv, srow, drow, buf0, buf1, acc, sem0, sem1):
    c = lax.axis_index("c")
    s = lax.axis_index("s")
    base = jnp.where(c == 0, 0, K0)
    nch = jnp.where(c == 0, K0, K1)

    pltpu.sync_copy(pk_hbm.at[s, pl.ds(base, KMAX)], pkv)

    def unpack_src(ch, b):
        for k in range(8):
            v = pkv[ch, pl.ds(16 * k, 16)]
            srow[b, pl.ds(16 * k, 16)] = lax.shift_right_logical(v, 14)

    def unpack_dst(ch, b):
        for k in range(8):
            v = pkv[ch, pl.ds(16 * k, 16)]
            drow[b, pl.ds(16 * k, 16)] = lax.bitwise_and(v, 16383)

    zv = jnp.zeros((16,), F32)

    def zrow(r, carry):
        for k in range(8):
            buf0[r, pl.ds(16 * k, 16)] = zv
        return carry

    lax.fori_loop(0, C, zrow, 0)
    row0 = s * RPT
    for t in range(RPT // C):
        pltpu.sync_copy(buf0, acc.at[pl.ds(row0 + t * C, C)])

    plsc.subcore_barrier()

    bufs = (buf0, buf1)
    sems = (sem0, sem1)
    for b in range(2):
        unpack_src(b, b)
        pltpu.async_copy(table.at[srow.at[b]], bufs[b], sems[b])

    def chunk_pair(i, carry):
        j = i * 2
        for b in range(2):
            ch = j + b
            pltpu.make_async_copy(table.at[srow.at[b]], bufs[b], sems[b]).wait()
            unpack_dst(ch, b)
            pltpu.sync_copy(bufs[b], acc.at[drow.at[b]], add=True)
            nxt = ch + 2

            @pl.when(nxt < nch)
            def _():
                unpack_src(nxt, b)
                pltpu.async_copy(table.at[srow.at[b]], bufs[b], sems[b])

        return carry

    lax.fori_loop(0, nch // 2, chunk_pair, 0)
    plsc.subcore_barrier()

    for t in range(RPT // C):
        r0 = row0 + t * C
        pltpu.sync_copy(acc.at[pl.ds(r0, C)], buf0)
        pltpu.sync_copy(buf0, out_p.at[c, pl.ds(r0, C)])


@functools.partial(
    pl.kernel,
    out_type=jax.ShapeDtypeStruct((NW, N_ACC), F32),
    mesh=_mesh,
    compiler_params=pltpu.CompilerParams(needs_layout_passes=False),
    scratch_types=[
        pltpu.VMEM((NCH, C), jnp.int32),
        pltpu.VMEM((N_ACC,), F32),
    ],
)
def _cnt(pk_hbm, out_cnt, pkv, cntt):
    c = lax.axis_index("c")
    s = lax.axis_index("s")
    wid = c * NS + s
    pltpu.sync_copy(pk_hbm.at[wid], pkv)

    zv = jnp.zeros((16,), F32)

    def zrow(i, carry):
        cntt[pl.ds(16 * i, 16)] = zv
        return carry

    lax.fori_loop(0, N_ACC // 16, zrow, 0)
    ov = jnp.ones((16,), F32)

    def chunk(ch, carry):
        for k in range(8):
            v = pkv[ch, pl.ds(16 * k, 16)]
            dst = lax.bitwise_and(v, 16383)
            plsc.addupdate_scatter(cntt, [dst], ov)
        return carry

    lax.fori_loop(0, NCH, chunk, 0)
    pltpu.sync_copy(cntt, out_cnt.at[wid])


_R = 2048


def _tc1_body(p_ref, cnt_ref, x_ref, wlt_ref, bl_ref, wrt_ref,
              bns_ref, bnb_ref, h_ref):
    sagg = p_ref[0] + p_ref[1]
    cnt = jnp.sum(cnt_ref[...], axis=0)[:, None]
    mean = sagg / jnp.maximum(cnt, 1.0)
    h = jnp.dot(mean, wlt_ref[...], preferred_element_type=F32) + bl_ref[...]
    h = h + jnp.dot(x_ref[...], wrt_ref[...], preferred_element_type=F32)
    h = h * bns_ref[...] + bnb_ref[...]
    h_ref[...] = jnp.maximum(h, 0.0)


_tc1 = pl.pallas_call(
    _tc1_body,
    grid=(N_ACC // _R,),
    in_specs=[
        pl.BlockSpec((NC, _R, D), lambda i: (0, i, 0)),
        pl.BlockSpec((NW, _R), lambda i: (0, i)),
        pl.BlockSpec((_R, D), lambda i: (i, 0)),
        pl.BlockSpec((D, D), lambda i: (0, 0)),
        pl.BlockSpec((1, D), lambda i: (0, 0)),
        pl.BlockSpec((D, D), lambda i: (0, 0)),
        pl.BlockSpec((1, D), lambda i: (0, 0)),
        pl.BlockSpec((1, D), lambda i: (0, 0)),
    ],
    out_specs=pl.BlockSpec((_R, D), lambda i: (i, 0)),
    out_shape=jax.ShapeDtypeStruct((N_ACC, D), F32),
)


def _tc2_body(p_ref, cnt_ref, h1_ref, wlt_ref, bl_ref, wrt_ref,
              bns_ref, bnb_ref, wc1t_ref, bc1_ref, wc2t_ref, bc2_ref,
              lg_ref, pr_ref):
    sagg = p_ref[0] + p_ref[1]
    cnt = jnp.sum(cnt_ref[...], axis=0)[:, None]
    mean = sagg / jnp.maximum(cnt, 1.0)
    h = jnp.dot(mean, wlt_ref[...], preferred_element_type=F32) + bl_ref[...]
    h = h + jnp.dot(h1_ref[...], wrt_ref[...], preferred_element_type=F32)
    h = jnp.maximum(h * bns_ref[...] + bnb_ref[...], 0.0)
    z = jnp.maximum(
        jnp.dot(h, wc1t_ref[...], preferred_element_type=F32) + bc1_ref[...],
        0.0)
    lg = jnp.dot(z, wc2t_ref[...], preferred_element_type=F32) + bc2_ref[...]
    m = jnp.max(lg, axis=-1, keepdims=True)
    ex = jnp.exp(lg - m)
    pr = ex / jnp.sum(ex, axis=-1, keepdims=True)
    lg_ref[...] = lg
    pr_ref[...] = pr


_tc2 = pl.pallas_call(
    _tc2_body,
    grid=(N_ACC // _R,),
    in_specs=[
        pl.BlockSpec((NC, _R, D), lambda i: (0, i, 0)),
        pl.BlockSpec((NW, _R), lambda i: (0, i)),
        pl.BlockSpec((_R, D), lambda i: (i, 0)),
        pl.BlockSpec((D, D), lambda i: (0, 0)),
        pl.BlockSpec((1, D), lambda i: (0, 0)),
        pl.BlockSpec((D, D), lambda i: (0, 0)),
        pl.BlockSpec((1, D), lambda i: (0, 0)),
        pl.BlockSpec((1, D), lambda i: (0, 0)),
        pl.BlockSpec((D, 64), lambda i: (0, 0)),
        pl.BlockSpec((1, 64), lambda i: (0, 0)),
        pl.BlockSpec((64, D), lambda i: (0, 0)),
        pl.BlockSpec((1, D), lambda i: (0, 0)),
    ],
    out_specs=(pl.BlockSpec((_R, D), lambda i: (i, 0)),
               pl.BlockSpec((_R, D), lambda i: (i, 0))),
    out_shape=(jax.ShapeDtypeStruct((N_ACC, D), F32),
               jax.ShapeDtypeStruct((N_ACC, D), F32)),
)


def kernel(x, edge_index, Wl1, bl1, Wr1, g1, be1,
           Wl2, bl2, Wr2, g2, be2, Wc1, bc1, Wc2, bc2):
    src = edge_index[0]
    dst = edge_index[1]
    pad = E_PAD - E
    packed = src * 16384 + dst
    pk_flat = jnp.concatenate([packed, jnp.full((pad,), N, jnp.int32)])
    pk32 = pk_flat.reshape(NW, NCH, C)
    pk16 = pk_flat.reshape(NS, NCHT, C)
    x_pad = jnp.zeros((N_ACC, D), F32).at[:N].set(x)

    p1 = _agg(x_pad, pk16)
    cnt16 = _cnt(pk32)

    bscale = 1.0 / jnp.sqrt(jnp.float32(1.0 + 1e-5))
    h1 = _tc1(p1, cnt16, x_pad, Wl1.T, bl1.reshape(1, D), Wr1.T,
              (g1 * bscale).reshape(1, D), be1.reshape(1, D))

    p2 = _agg(h1, pk16)

    wc2t = jnp.zeros((64, D), F32).at[:, :2].set(Wc2.T)
    bc2p = jnp.full((1, D), -1e30, F32).at[0, :2].set(bc2)
    logits_pad, probs_pad = _tc2(
        p2, cnt16, h1, Wl2.T, bl2.reshape(1, D), Wr2.T,
        (g2 * bscale).reshape(1, D), be2.reshape(1, D),
        Wc1.T, bc1.reshape(1, 64), wc2t, bc2p)

    return (logits_pad[:N, :2], probs_pad[:N, :2])
